# Initial kernel scaffold; baseline (speedup 1.0000x reference)
import jax
import jax.numpy as jnp
from jax.experimental import pallas as pl


def kernel(x, edge_index, Wc0, bc0, Wa0, ba0, Wd0, bd0, Wc1, bc1, Wa1, ba1, Wd1, bd1, Wc2, bc2, Wa2, ba2, Wd2, bd2, g0, be0, rm0, rv0, g1, be1, rm1, rv1):
    raise NotImplementedError("write your pallas kernel here")


# speedup vs baseline: 15.9775x; 15.9775x over previous
import functools

import jax
import jax.numpy as jnp
from jax import lax
from jax.experimental import pallas as pl
from jax.experimental.pallas import tpu as pltpu
from jax.experimental.pallas import tpu_sc as plsc

_N = 10000
_E = 320000
_H = 128
_K = 100

_NC = 2
_NS = 16
_NW = _NC * _NS
_EPW = _E // _NW
_CH = 80
_NCHUNK = _EPW // _CH
_RPB = 624
_TAIL = _N - _NS * _RPB

_mesh = plsc.VectorSubcoreMesh(core_axis_name="c", subcore_axis_name="s")


@functools.partial(
    pl.kernel,
    out_type=jax.ShapeDtypeStruct((_NC, _N, 16), jnp.float32),
    mesh=_mesh,
    scratch_types=[
        pltpu.VMEM((_NCHUNK, _CH), jnp.int32),
        pltpu.VMEM((_CH, 16), jnp.float32),
        pltpu.VMEM_SHARED((_N, 16), jnp.float32),
    ],
)
def _deg_kernel(col3_hbm, zeros_hbm, ones_hbm, out_hbm, cidx, ones_v, deg_sh):
    cid = lax.axis_index("c")
    sid = lax.axis_index("s")
    wid = cid * _NS + sid
    pltpu.sync_copy(zeros_hbm.at[pl.ds(0, _RPB)],
                    deg_sh.at[pl.ds(sid * _RPB, _RPB)])

    @pl.when(sid == _NS - 1)
    def _():
        pltpu.sync_copy(zeros_hbm.at[pl.ds(0, _TAIL)],
                        deg_sh.at[pl.ds(_NS * _RPB, _TAIL)])

    pltpu.sync_copy(ones_hbm, ones_v)
    pltpu.sync_copy(col3_hbm.at[wid], cidx)
    plsc.subcore_barrier()

    def body(j, carry):
        pltpu.sync_copy(ones_v, deg_sh.at[cidx.at[j]], add=True)
        return carry

    lax.fori_loop(0, _NCHUNK, body, 0)
    plsc.subcore_barrier()
    pltpu.sync_copy(deg_sh.at[pl.ds(sid * _RPB, _RPB)],
                    out_hbm.at[cid].at[pl.ds(sid * _RPB, _RPB)])

    @pl.when(sid == _NS - 1)
    def _():
        pltpu.sync_copy(deg_sh.at[pl.ds(_NS * _RPB, _TAIL)],
                        out_hbm.at[cid].at[pl.ds(_NS * _RPB, _TAIL)])


@functools.partial(
    pl.kernel,
    out_type=jax.ShapeDtypeStruct((_NC, _N, _H), jnp.float32),
    mesh=_mesh,
    scratch_types=[
        pltpu.VMEM((_NCHUNK, _CH), jnp.int32),
        pltpu.VMEM((_NCHUNK, _CH), jnp.int32),
        pltpu.VMEM((_CH, _H), jnp.float32),
        pltpu.VMEM_SHARED((_N, _H), jnp.float32),
        pltpu.SemaphoreType.DMA,
    ],
)
def _edge_kernel(y_hbm, row3_hbm, col3_hbm, zeros_hbm, out_hbm,
                 ridx, cidx, rows, z_sh, gsem):
    cid = lax.axis_index("c")
    sid = lax.axis_index("s")
    wid = cid * _NS + sid
    pltpu.sync_copy(zeros_hbm.at[pl.ds(0, _RPB)],
                    z_sh.at[pl.ds(sid * _RPB, _RPB)])

    @pl.when(sid == _NS - 1)
    def _():
        pltpu.sync_copy(zeros_hbm.at[pl.ds(0, _TAIL)],
                        z_sh.at[pl.ds(_NS * _RPB, _TAIL)])

    pltpu.sync_copy(row3_hbm.at[wid], ridx)
    pltpu.sync_copy(col3_hbm.at[wid], cidx)
    plsc.subcore_barrier()

    def body(j, carry):
        pltpu.async_copy(y_hbm.at[ridx.at[j]], rows, gsem).wait()
        pltpu.sync_copy(rows, z_sh.at[cidx.at[j]], add=True)
        return carry

    lax.fori_loop(0, _NCHUNK, body, 0)
    plsc.subcore_barrier()
    pltpu.sync_copy(z_sh.at[pl.ds(sid * _RPB, _RPB)],
                    out_hbm.at[cid].at[pl.ds(sid * _RPB, _RPB)])

    @pl.when(sid == _NS - 1)
    def _():
        pltpu.sync_copy(z_sh.at[pl.ds(_NS * _RPB, _TAIL)],
                        out_hbm.at[cid].at[pl.ds(_NS * _RPB, _TAIL)])


_BN = 2000
_NBLK = _N // _BN


def _dis_from(degp_ref):
    dp = degp_ref[0, :, 0:1] + degp_ref[1, :, 0:1] + 1.0
    return 1.0 / jnp.sqrt(dp)


def _mm(a, b):
    return jnp.dot(a, b, preferred_element_type=jnp.float32)


def _lra_prep(h, waU, waV, waZ, waT, baU, baV, baZ, baT,
              U_ref, T_ref, M_ref, us_ref, vs_ref):
    U = jnp.maximum(_mm(h, waU) + baU, 0.0)
    V = jnp.maximum(_mm(h, waV) + baV, 0.0)
    Z = jnp.maximum(_mm(h, waZ) + baZ, 0.0)
    T = jnp.maximum(_mm(h, waT) + baT, 0.0)
    U_ref[...] = U
    T_ref[...] = T

    @pl.when(pl.program_id(0) == 0)
    def _():
        M_ref[...] = jnp.zeros_like(M_ref)
        us_ref[...] = jnp.zeros_like(us_ref)
        vs_ref[...] = jnp.zeros_like(vs_ref)

    M_ref[...] += lax.dot_general(V, Z, (((0,), (0,)), ((), ())),
                                  preferred_element_type=jnp.float32)
    us_ref[...] += jnp.sum(U, axis=0, keepdims=True)
    vs_ref[...] += jnp.sum(V, axis=0, keepdims=True)


def _prep_body(x_ref, degp_ref, wc_ref,
               waU_ref, waV_ref, waZ_ref, waT_ref,
               baU_ref, baV_ref, baZ_ref, baT_ref,
               y_ref, U_ref, T_ref, M_ref, us_ref, vs_ref):
    x = x_ref[...]
    dis = _dis_from(degp_ref)
    y_ref[...] = _mm(x, wc_ref[...]) * dis
    _lra_prep(x, waU_ref[...], waV_ref[...], waZ_ref[...], waT_ref[...],
              baU_ref[...], baV_ref[...], baZ_ref[...], baT_ref[...],
              U_ref, T_ref, M_ref, us_ref, vs_ref)


def _combine(zp_ref, y_ref, degp_ref, U_ref, T_ref, M_ref, us_ref, vs_ref,
             wdA_ref, wdB_ref, wdC_ref, bd_ref, bc_ref):
    dis = _dis_from(degp_ref)
    z = zp_ref[0] + zp_ref[1] + y_ref[...]
    xl = jnp.maximum(z * dis + bc_ref[...], 0.0)
    nf = jnp.sum(us_ref[...] * vs_ref[...]) * (1.0 / _N) + 1e-6
    MW = _mm(M_ref[...], wdA_ref[...])
    h = (_mm(U_ref[...], MW) * (1.0 / nf) + _mm(T_ref[...], wdB_ref[...])
         + _mm(xl, wdC_ref[...]) + bd_ref[...])
    return h, dis


def _mid_body(zp_ref, y_ref, degp_ref, U_ref, T_ref, M_ref, us_ref, vs_ref,
              wdA_ref, wdB_ref, wdC_ref, bd_ref, bc_ref,
              g_ref, be_ref, rm_ref, rv_ref,
              wc_ref, waU_ref, waV_ref, waZ_ref, waT_ref,
              baU_ref, baV_ref, baZ_ref, baT_ref,
              y2_ref, U2_ref, T2_ref, M2_ref, us2_ref, vs2_ref):
    h, dis = _combine(zp_ref, y_ref, degp_ref, U_ref, T_ref, M_ref,
                      us_ref, vs_ref, wdA_ref, wdB_ref, wdC_ref,
                      bd_ref, bc_ref)
    h = jnp.maximum(h, 0.0)
    h = ((h - rm_ref[...]) / jnp.sqrt(rv_ref[...] + 1e-5)
         * g_ref[...] + be_ref[...])
    y2_ref[...] = _mm(h, wc_ref[...]) * dis
    _lra_prep(h, waU_ref[...], waV_ref[...], waZ_ref[...], waT_ref[...],
              baU_ref[...], baV_ref[...], baZ_ref[...], baT_ref[...],
              U2_ref, T2_ref, M2_ref, us2_ref, vs2_ref)


def _final_body(zp_ref, y_ref, degp_ref, U_ref, T_ref, M_ref, us_ref, vs_ref,
                wdA_ref, wdB_ref, wdC_ref, bd_ref, bc_ref, out_ref):
    h, _ = _combine(zp_ref, y_ref, degp_ref, U_ref, T_ref, M_ref,
                    us_ref, vs_ref, wdA_ref, wdB_ref, wdC_ref,
                    bd_ref, bc_ref)
    out_ref[...] = h


_f32 = jnp.float32
_PREP_OUT = [
    jax.ShapeDtypeStruct((_N, _H), _f32),
    jax.ShapeDtypeStruct((_N, _K), _f32),
    jax.ShapeDtypeStruct((_N, _K), _f32),
    jax.ShapeDtypeStruct((_K, _K), _f32),
    jax.ShapeDtypeStruct((1, _K), _f32),
    jax.ShapeDtypeStruct((1, _K), _f32),
]


def _c(shape):
    return pl.BlockSpec(shape, lambda i: (0,) * len(shape))


_S_ROWH = pl.BlockSpec((_BN, _H), lambda i: (i, 0))
_S_ROWK = pl.BlockSpec((_BN, _K), lambda i: (i, 0))
_S_ZP = pl.BlockSpec((_NC, _BN, _H), lambda i: (0, i, 0))
_S_DEGP = pl.BlockSpec((_NC, _BN, 16), lambda i: (0, i, 0))
_A_SPECS = [_c((_H, _K))] * 4 + [_c((1, _K))] * 4
_D_SPECS = [_c((_K, _H)), _c((_K, _H)), _c((_H, _H)), _c((1, _H))]
_PREP_OUT_SPECS = [_S_ROWH, _S_ROWK, _S_ROWK,
                   _c((_K, _K)), _c((1, _K)), _c((1, _K))]
_COMBINE_IN_SPECS = ([_S_ZP, _S_ROWH, _S_DEGP, _S_ROWK, _S_ROWK,
                      _c((_K, _K)), _c((1, _K)), _c((1, _K))]
                     + _D_SPECS + [_c((1, _H))])


def kernel(x, edge_index,
           Wc0, bc0, Wa0, ba0, Wd0, bd0,
           Wc1, bc1, Wa1, ba1, Wd1, bd1,
           Wc2, bc2, Wa2, ba2, Wd2, bd2,
           g0, be0, rm0, rv0,
           g1, be1, rm1, rv1):
    row3 = edge_index[0].reshape(_NW, _NCHUNK, _CH)
    col3 = edge_index[1].reshape(_NW, _NCHUNK, _CH)
    zeros16 = jnp.zeros((_RPB, 16), _f32)
    zerosH = jnp.zeros((_RPB, _H), _f32)
    ones16 = jnp.ones((_CH, 16), _f32)

    degp = _deg_kernel(col3, zeros16, ones16)

    def split_a(Wa, ba):
        b = ba.reshape(1, -1)
        return (Wa[:, :_K], Wa[:, _K:2 * _K], Wa[:, 2 * _K:3 * _K],
                Wa[:, 3 * _K:], b[:, :_K], b[:, _K:2 * _K],
                b[:, 2 * _K:3 * _K], b[:, 3 * _K:])

    def split_d(Wd, bd):
        return Wd[:_K], Wd[_K:2 * _K], Wd[2 * _K:], bd.reshape(1, -1)

    a0 = split_a(Wa0, ba0)
    a1 = split_a(Wa1, ba1)
    a2 = split_a(Wa2, ba2)
    d0 = split_d(Wd0, bd0)
    d1 = split_d(Wd1, bd1)
    d2 = split_d(Wd2, bd2)
    bn0 = (g0.reshape(1, -1), be0.reshape(1, -1),
           rm0.reshape(1, -1), rv0.reshape(1, -1))
    bn1 = (g1.reshape(1, -1), be1.reshape(1, -1),
           rm1.reshape(1, -1), rv1.reshape(1, -1))

    y0, U0, T0, M0, us0, vs0 = pl.pallas_call(
        _prep_body, grid=(_NBLK,),
        in_specs=[_S_ROWH, _S_DEGP, _c((_H, _H))] + _A_SPECS,
        out_specs=_PREP_OUT_SPECS,
        out_shape=_PREP_OUT)(x, degp, Wc0, *a0)

    zp0 = _edge_kernel(y0, row3, col3, zerosH)
    y1, U1, T1, M1, us1, vs1 = pl.pallas_call(
        _mid_body, grid=(_NBLK,),
        in_specs=(_COMBINE_IN_SPECS
                  + [_c((1, _H))] * 4 + [_c((_H, _H))] + _A_SPECS),
        out_specs=_PREP_OUT_SPECS,
        out_shape=_PREP_OUT)(
            zp0, y0, degp, U0, T0, M0, us0, vs0,
            *d0, bc0.reshape(1, -1), *bn0, Wc1, *a1)

    zp1 = _edge_kernel(y1, row3, col3, zerosH)
    y2, U2, T2, M2, us2, vs2 = pl.pallas_call(
        _mid_body, grid=(_NBLK,),
        in_specs=(_COMBINE_IN_SPECS
                  + [_c((1, _H))] * 4 + [_c((_H, _H))] + _A_SPECS),
        out_specs=_PREP_OUT_SPECS,
        out_shape=_PREP_OUT)(
            zp1, y1, degp, U1, T1, M1, us1, vs1,
            *d1, bc1.reshape(1, -1), *bn1, Wc2, *a2)

    zp2 = _edge_kernel(y2, row3, col3, zerosH)
    out = pl.pallas_call(
        _final_body, grid=(_NBLK,),
        in_specs=_COMBINE_IN_SPECS,
        out_specs=_S_ROWH,
        out_shape=jax.ShapeDtypeStruct((_N, _H), _f32))(
            zp2, y2, degp, U2, T2, M2, us2, vs2,
            *d2, bc2.reshape(1, -1))
    return out
